# baseline (speedup 1.0000x reference)
import functools

import jax
import jax.numpy as jnp
from jax import lax
from jax.experimental import pallas as pl
from jax.experimental.pallas import tpu as pltpu
from jax.experimental.pallas import tpu_sc as plsc

VOCAB = 1000000
EMBED = 300
BATCH = 1024
SEQ = 512
HIDDEN = 16

NC = 2
NS = 16
NW = NC * NS
ROWS_PER_W = BATCH // NW
CHUNK = 128
CHUNKS_PER_ROW = SEQ // CHUNK
CHUNKS_PER_W = ROWS_PER_W * CHUNKS_PER_ROW

P_BLOCK = 16384


def _sc_pool_body(tok_hbm, p_hbm, out_hbm, tok_v, idx_v, buf_v, acc_v, sems):
    wid = lax.axis_index("s") * NC + lax.axis_index("c")
    pltpu.sync_copy(tok_hbm.at[pl.ds(wid * CHUNKS_PER_W, CHUNKS_PER_W)], tok_v)

    def copy_idx(c, parity):
        for v in range(8):
            idx_v[parity, pl.ds(16 * v, 16)] = tok_v[c, pl.ds(16 * v, 16)]

    def start_gather(parity):
        pltpu.make_async_copy(p_hbm.at[idx_v.at[parity]], buf_v.at[parity],
                              sems.at[parity]).start()

    def wait_gather(parity):
        pltpu.make_async_copy(p_hbm.at[idx_v.at[parity]], buf_v.at[parity],
                              sems.at[parity]).wait()

    copy_idx(0, 0)
    start_gather(0)

    def chunk_body(c, parity):
        @pl.when(c < CHUNKS_PER_W - 1)
        def _():
            copy_idx(c + 1, 1 - parity)
            start_gather(1 - parity)
        wait_gather(parity)
        racc = c // CHUNKS_PER_ROW

        def accum8(r, acc):
            for rr in range(8):
                acc = acc + buf_v[parity, 8 * r + rr, pl.ds(0, 16)]
            return acc

        acc = lax.fori_loop(0, CHUNK // 8, accum8,
                            jnp.zeros((16,), jnp.float32))
        acc_v[racc, pl.ds(0, 16)] = acc_v[racc, pl.ds(0, 16)] + acc

    def pair_body(g, carry):
        chunk_body(2 * g, 0)
        chunk_body(2 * g + 1, 1)
        return carry

    def zero_row(i, carry):
        acc_v[i, pl.ds(0, 16)] = jnp.zeros((16,), jnp.float32)
        return carry
    lax.fori_loop(0, ROWS_PER_W, zero_row, 0)

    lax.fori_loop(0, CHUNKS_PER_W // 2, pair_body, 0)

    pltpu.sync_copy(acc_v, out_hbm.at[pl.ds(wid * ROWS_PER_W, ROWS_PER_W)])


_sc_pool = functools.partial(
    pl.kernel,
    mesh=plsc.VectorSubcoreMesh(core_axis_name="c", subcore_axis_name="s"),
    out_type=jax.ShapeDtypeStruct((BATCH, HIDDEN), jnp.float32),
    scratch_types=[
        pltpu.VMEM((CHUNKS_PER_W, CHUNK), jnp.int32),
        pltpu.VMEM((2, CHUNK), jnp.int32),
        pltpu.VMEM((2, CHUNK, HIDDEN), jnp.float32),
        pltpu.VMEM((ROWS_PER_W, HIDDEN), jnp.float32),
        pltpu.SemaphoreType.DMA((2,)),
    ],
    compiler_params=pltpu.CompilerParams(use_tc_tiling_on_sc=False),
)(_sc_pool_body)


def _proj_body(xt_ref, w_ref, o_ref):
    val = lax.dot_general(
        xt_ref[...].astype(jnp.bfloat16), w_ref[...].astype(jnp.bfloat16),
        (((0,), (0,)), ((), ())), preferred_element_type=jnp.float32)
    v3 = val.reshape(P_BLOCK // 8, 8, 128)
    lane = lax.broadcasted_iota(jnp.int32, (P_BLOCK // 8, 128), 1)
    out = jnp.zeros((P_BLOCK // 8, 128), jnp.float32)
    for a in range(8):
        va = lax.squeeze(lax.slice_in_dim(v3, a, a + 1, axis=1), (1,))
        out = jnp.where((lane >> 4) == a, va, out)
    o_ref[...] = out


def _mlp_body(x_ref, b1_ref, w2_ref, b2_ref, o_ref):
    h = jnp.maximum(x_ref[...] + b1_ref[...], 0.0)
    o = jnp.dot(h, w2_ref[...], preferred_element_type=jnp.float32)
    o_ref[...] = jax.nn.sigmoid(o + b2_ref[...])


def kernel(tokens, emb_table, W1, b1, W2, b2):
    tok = tokens.reshape(BATCH * CHUNKS_PER_ROW, CHUNK)

    embt = emb_table.T
    w1rep = jnp.tile(W1 * (1.0 / SEQ), (1, 128 // HIDDEN))
    p = pl.pallas_call(
        _proj_body,
        grid=((VOCAB + P_BLOCK - 1) // P_BLOCK,),
        in_specs=[
            pl.BlockSpec((EMBED, P_BLOCK), lambda i: (0, i)),
            pl.BlockSpec((EMBED, 128), lambda i: (0, 0)),
        ],
        out_specs=pl.BlockSpec((P_BLOCK // 8, 128), lambda i: (i, 0)),
        out_shape=jax.ShapeDtypeStruct((VOCAB // 8, 128), jnp.float32),
    )(embt, w1rep)

    sums = _sc_pool(tok, p.reshape(VOCAB, HIDDEN))

    out = pl.pallas_call(
        _mlp_body,
        out_shape=jax.ShapeDtypeStruct((BATCH, 1), jnp.float32),
    )(sums, b1.reshape(1, HIDDEN), W2, b2.reshape(1, 1))
    return out


# speedup vs baseline: 1.0136x; 1.0136x over previous
import functools

import jax
import jax.numpy as jnp
from jax import lax
from jax.experimental import pallas as pl
from jax.experimental.pallas import tpu as pltpu
from jax.experimental.pallas import tpu_sc as plsc

VOCAB = 1000000
EMBED = 300
BATCH = 1024
SEQ = 512
HIDDEN = 16

NC = 2
NS = 16
NW = NC * NS
ROWS_PER_W = BATCH // NW
CHUNK = 128
CHUNKS_PER_ROW = SEQ // CHUNK
CHUNKS_PER_W = ROWS_PER_W * CHUNKS_PER_ROW

P_BLOCK = 18432


def _sc_pool_body(tok_hbm, p_hbm, out_hbm, tok_v, idx_v, buf_v, acc_v, sems):
    wid = lax.axis_index("s") * NC + lax.axis_index("c")
    pltpu.sync_copy(tok_hbm.at[pl.ds(wid * CHUNKS_PER_W, CHUNKS_PER_W)], tok_v)

    def copy_idx(c, parity):
        for v in range(8):
            idx_v[parity, pl.ds(16 * v, 16)] = tok_v[c, pl.ds(16 * v, 16)]

    def start_gather(parity):
        pltpu.make_async_copy(p_hbm.at[idx_v.at[parity]], buf_v.at[parity],
                              sems.at[parity]).start()

    def wait_gather(parity):
        pltpu.make_async_copy(p_hbm.at[idx_v.at[parity]], buf_v.at[parity],
                              sems.at[parity]).wait()

    copy_idx(0, 0)
    start_gather(0)

    def chunk_body(c, parity):
        @pl.when(c < CHUNKS_PER_W - 1)
        def _():
            copy_idx(c + 1, 1 - parity)
            start_gather(1 - parity)
        wait_gather(parity)
        racc = c // CHUNKS_PER_ROW

        def accum8(r, acc):
            for rr in range(8):
                acc = acc + buf_v[parity, 8 * r + rr, pl.ds(0, 16)]
            return acc

        acc = lax.fori_loop(0, CHUNK // 8, accum8,
                            jnp.zeros((16,), jnp.float32))
        acc_v[racc, pl.ds(0, 16)] = acc_v[racc, pl.ds(0, 16)] + acc

    def pair_body(g, carry):
        chunk_body(2 * g, 0)
        chunk_body(2 * g + 1, 1)
        return carry

    def zero_row(i, carry):
        acc_v[i, pl.ds(0, 16)] = jnp.zeros((16,), jnp.float32)
        return carry
    lax.fori_loop(0, ROWS_PER_W, zero_row, 0)

    lax.fori_loop(0, CHUNKS_PER_W // 2, pair_body, 0)

    pltpu.sync_copy(acc_v, out_hbm.at[pl.ds(wid * ROWS_PER_W, ROWS_PER_W)])


_sc_pool = functools.partial(
    pl.kernel,
    mesh=plsc.VectorSubcoreMesh(core_axis_name="c", subcore_axis_name="s"),
    out_type=jax.ShapeDtypeStruct((BATCH, HIDDEN), jnp.float32),
    scratch_types=[
        pltpu.VMEM((CHUNKS_PER_W, CHUNK), jnp.int32),
        pltpu.VMEM((2, CHUNK), jnp.int32),
        pltpu.VMEM((2, CHUNK, HIDDEN), jnp.float32),
        pltpu.VMEM((ROWS_PER_W, HIDDEN), jnp.float32),
        pltpu.SemaphoreType.DMA((2,)),
    ],
    compiler_params=pltpu.CompilerParams(use_tc_tiling_on_sc=False),
)(_sc_pool_body)


def _proj_body(xt_ref, w_ref, o_ref):
    val = lax.dot_general(
        xt_ref[...].astype(jnp.bfloat16), w_ref[...].astype(jnp.bfloat16),
        (((0,), (0,)), ((), ())), preferred_element_type=jnp.float32)
    v3 = val.reshape(P_BLOCK // 8, 8, 128)
    lane = lax.broadcasted_iota(jnp.int32, (P_BLOCK // 8, 128), 1)
    out = jnp.zeros((P_BLOCK // 8, 128), jnp.float32)
    for a in range(8):
        va = lax.squeeze(lax.slice_in_dim(v3, a, a + 1, axis=1), (1,))
        out = jnp.where((lane >> 4) == a, va, out)
    o_ref[...] = out


def _mlp_body(x_ref, b1_ref, w2_ref, b2_ref, o_ref):
    h = jnp.maximum(x_ref[...] + b1_ref[...], 0.0)
    o = jnp.dot(h, w2_ref[...], preferred_element_type=jnp.float32)
    o_ref[...] = jax.nn.sigmoid(o + b2_ref[...])


def kernel(tokens, emb_table, W1, b1, W2, b2):
    tok = tokens.reshape(BATCH * CHUNKS_PER_ROW, CHUNK)

    embt = emb_table.T
    w1rep = jnp.tile(W1 * (1.0 / SEQ), (1, 128 // HIDDEN))
    p = pl.pallas_call(
        _proj_body,
        grid=((VOCAB + P_BLOCK - 1) // P_BLOCK,),
        in_specs=[
            pl.BlockSpec((EMBED, P_BLOCK), lambda i: (0, i)),
            pl.BlockSpec((EMBED, 128), lambda i: (0, 0)),
        ],
        out_specs=pl.BlockSpec((P_BLOCK // 8, 128), lambda i: (i, 0)),
        out_shape=jax.ShapeDtypeStruct((VOCAB // 8, 128), jnp.float32),
    )(embt, w1rep)

    sums = _sc_pool(tok, p.reshape(VOCAB, HIDDEN))

    out = pl.pallas_call(
        _mlp_body,
        out_shape=jax.ShapeDtypeStruct((BATCH, 1), jnp.float32),
    )(sums, b1.reshape(1, HIDDEN), W2, b2.reshape(1, 1))
    return out
